# baseline (speedup 1.0000x reference)
import functools

import jax
import jax.numpy as jnp
from jax import lax
from jax.experimental import pallas as pl
from jax.experimental.pallas import tpu as pltpu
from jax.experimental.pallas import tpu_sc as plsc

VOCAB = 1000000
D = 64
T = 200
B = 4096

NC = 2
NS = 16
NW = NC * NS

ROWS = B * T
RPW = ROWS // NW
CH = 128
NCH = RPW // CH
NBUF = 4
PWORDS = (T + CH) * D + D

_mesh = plsc.VectorSubcoreMesh(core_axis_name="c", subcore_axis_name="s")

_scratch = []
for _ in range(NBUF):
    _scratch += [
        pltpu.VMEM((CH,), jnp.int32),
        pltpu.VMEM((CH,), jnp.int32),
        pltpu.VMEM((CH, 2 * D), jnp.float32),
        pltpu.VMEM((CH // 2, 2 * D), jnp.float32),
    ]
_scratch += [
    pltpu.VMEM((PWORDS,), jnp.float32),
    pltpu.SemaphoreType.DMA((NBUF,)),
    pltpu.SemaphoreType.DMA((NBUF,)),
    pltpu.SemaphoreType.DMA((NBUF,)),
]


@functools.partial(
    pl.kernel,
    mesh=_mesh,
    out_type=jax.ShapeDtypeStruct((ROWS // 2, 2 * D), jnp.float32),
    scratch_types=_scratch,
)
def _embed(x_hbm, xrow_hbm, tab2_hbm, pos2_hbm, out_hbm, *scr):
    xi = [scr[4 * b + 0] for b in range(NBUF)]
    xr = [scr[4 * b + 1] for b in range(NBUF)]
    pr = [scr[4 * b + 2] for b in range(NBUF)]
    ov = [scr[4 * b + 3] for b in range(NBUF)]
    pos_v, sem_i, sem_g, sem_o = scr[4 * NBUF:]

    wid = lax.axis_index("s") * NC + lax.axis_index("c")
    base = wid * RPW
    pltpu.sync_copy(pos2_hbm, pos_v)

    def group_body(g):
        for b in range(NBUF):
            cb = pl.multiple_of(base + (g + b) * CH, CH)
            pltpu.async_copy(x_hbm.at[pl.ds(cb, CH)], xi[b], sem_i.at[b])
            pltpu.async_copy(xrow_hbm.at[pl.ds(cb, CH)], xr[b], sem_i.at[b])
        for b in range(NBUF):
            cb = pl.multiple_of(base + (g + b) * CH, CH)
            pltpu.make_async_copy(x_hbm.at[pl.ds(cb, CH)], xi[b],
                                  sem_i.at[b]).wait()
            pltpu.make_async_copy(xrow_hbm.at[pl.ds(cb, CH)], xr[b],
                                  sem_i.at[b]).wait()
            pltpu.async_copy(tab2_hbm.at[xr[b]], pr[b], sem_g.at[b])
        for b in range(NBUF):
            cb = pl.multiple_of(base + (g + b) * CH, CH)
            t0 = lax.rem(cb, T)
            pltpu.make_async_copy(tab2_hbm.at[xr[b]], pr[b],
                                  sem_g.at[b]).wait()

            def rows16_body(m, carry, b=b, t0=t0):
                offv = (xi[b][pl.ds(m * 16, 16)] & 1) * D
                for r in range(16):
                    i = m * 16 + r
                    off = offv[r]
                    pbase = (t0 + i) * D
                    orow = m * 8 + (r // 2)
                    ohalf = (r & 1) * D
                    for j in range(D // 16):
                        src = pr[b][i, pl.ds(off + j * 16, 16)]
                        pos = pos_v[pl.ds(pbase + j * 16, 16)]
                        ov[b][orow, pl.ds(ohalf + j * 16, 16)] = src + pos
                return carry

            lax.fori_loop(0, CH // 16, rows16_body, 0)
            ob = pl.multiple_of(cb // 2, CH // 2)
            pltpu.async_copy(ov[b], out_hbm.at[pl.ds(ob, CH // 2)],
                             sem_o.at[b])
        for b in range(NBUF):
            cb = pl.multiple_of(base + (g + b) * CH, CH)
            ob = pl.multiple_of(cb // 2, CH // 2)
            pltpu.make_async_copy(ov[b], out_hbm.at[pl.ds(ob, CH // 2)],
                                  sem_o.at[b]).wait()

    pl.loop(0, NCH, step=NBUF)(group_body)


def kernel(x, text_embedding, positional_embedding):
    xf = x.reshape(-1).astype(jnp.int32)
    xrow = lax.shift_right_logical(xf, 1)
    tab2 = text_embedding.reshape(VOCAB // 2, 2 * D)
    pflat = positional_embedding.reshape(-1)
    pos2 = jnp.concatenate([pflat, pflat[: PWORDS - T * D]])
    out = _embed(xf, xrow, tab2, pos2)
    return out.reshape(B, T, D)
